# baseline (speedup 1.0000x reference)
import functools

import jax
import jax.numpy as jnp
from jax import lax
from jax.experimental import pallas as pl
from jax.experimental.pallas import tpu as pltpu
from jax.experimental.pallas import tpu_sc as plsc


_NW = 32
_CH = 128


def _sc_gather(table, idx):
    nrows = idx.shape[0]
    h = table.shape[1]
    assert nrows % _CH == 0
    nchunk = nrows // _CH
    per = (nchunk + _NW - 1) // _NW
    mesh = plsc.VectorSubcoreMesh(core_axis_name="c", subcore_axis_name="s")

    @functools.partial(
        pl.kernel,
        mesh=mesh,
        out_type=jax.ShapeDtypeStruct((nrows, h), table.dtype),
        scratch_types=[
            pltpu.VMEM((_CH,), jnp.int32),
            pltpu.VMEM((_CH, h), table.dtype),
            pltpu.SemaphoreType.DMA,
        ],
    )
    def gk(table_hbm, idx_hbm, out_hbm, idx_v, rows_v, sem):
        wid = lax.axis_index("s") * 2 + lax.axis_index("c")

        def body(j, carry):
            c = j * _NW + wid

            @pl.when(c < nchunk)
            def _():
                base = c * _CH
                pltpu.sync_copy(idx_hbm.at[pl.ds(base, _CH)], idx_v)
                pltpu.async_copy(table_hbm.at[idx_v], rows_v, sem).wait()
                pltpu.sync_copy(rows_v, out_hbm.at[pl.ds(base, _CH)])

            return carry

        lax.fori_loop(0, per, body, 0)

    return gk(table, idx)


_NB = 400


def _gelu(x):
    return 0.5 * x * (1.0 + lax.erf(x * 0.7071067811865476))


def _bdot(x, w):
    return jnp.dot(x.astype(jnp.bfloat16), w, preferred_element_type=jnp.float32)


def _ln(x, s, o):
    mu = jnp.mean(x, axis=-1, keepdims=True)
    xc = x - mu
    var = jnp.mean(xc * xc, axis=-1, keepdims=True)
    return s * xc * lax.rsqrt(var + 1e-5) + o


def _proj_body(hv_ref, w1a_ref, w1c_ref, b1_ref, a_ref, p_ref):
    hv = hv_ref[...]
    a_ref[...] = _bdot(hv, w1a_ref[...]) + b1_ref[...]
    p_ref[...] = _bdot(hv, w1c_ref[...])


def _enc_body(hv_ref, he_ref, c1_ref, a1_ref, ma_ref, mv_ref,
              w1b_ref, w2_ref, b2_ref, w3_ref, b3_ref,
              win_ref, bin_ref, wout_ref, bout_ref,
              n1s_ref, n1o_ref, n2s_ref, n2o_ref,
              w11a_ref, b11_ref, w11c_ref,
              hvout_ref, a2_ref, p2_ref):
    hv = hv_ref[...]
    he = he_ref[...]
    nb, h = hv.shape
    eb = he.shape[0]
    k = eb // nb

    x = _bdot(he, w1b_ref[...]) + c1_ref[...].astype(jnp.float32)
    x = x.reshape(nb, k, h) + a1_ref[...][:, None, :]
    m = _gelu(x).reshape(eb, h)
    m = _gelu(_bdot(m, w2_ref[...]) + b2_ref[...])
    m = _bdot(m, w3_ref[...]) + b3_ref[...]
    m = m.reshape(nb, k, h) * ma_ref[...][:, :, None]
    dh = jnp.sum(m, axis=1) * (1.0 / 30.0)

    h1 = _ln(hv + dh, n1s_ref[...], n1o_ref[...])
    f = _gelu(_bdot(h1, win_ref[...]) + bin_ref[...])
    f = _bdot(f, wout_ref[...]) + bout_ref[...]
    h2 = _ln(h1 + f, n2s_ref[...], n2o_ref[...])
    hv_new = mv_ref[...] * h2

    hvout_ref[...] = hv_new
    a2_ref[...] = _bdot(hv_new, w11a_ref[...]) + b11_ref[...]
    p2_ref[...] = _bdot(hv_new, w11c_ref[...])


def _edge_body(he_ref, c2_ref, a2_ref,
               w11b_ref, w12_ref, b12_ref, w13_ref, b13_ref,
               n3s_ref, n3o_ref, heout_ref):
    he = he_ref[...]
    eb, h = he.shape
    a2 = a2_ref[...]
    nb = a2.shape[0]
    k = eb // nb

    x = _bdot(he, w11b_ref[...]) + c2_ref[...].astype(jnp.float32)
    x = x.reshape(nb, k, h) + a2[:, None, :]
    m = _gelu(x).reshape(eb, h)
    m = _gelu(_bdot(m, w12_ref[...]) + b12_ref[...])
    m = _bdot(m, w13_ref[...]) + b13_ref[...]
    heout_ref[...] = _ln(he + m, n3s_ref[...], n3o_ref[...])


def _full(shape):
    return pl.BlockSpec(shape, lambda i: (0, 0))


@jax.jit
def kernel(h_V, h_E, mask_V, mask_attend,
           W1_w, W1_b, W2_w, W2_b, W3_w, W3_b,
           W11_w, W11_b, W12_w, W12_b, W13_w, W13_b,
           Win_w, Win_b, Wout_w, Wout_b,
           n1_s, n1_o, n2_s, n2_o, n3_s, n3_o, E_idx):
    b, n, h = h_V.shape
    k = E_idx.shape[-1]
    ff = Win_w.shape[1]
    nb = _NB
    grid = n // nb
    eb = nb * k

    hv = h_V.reshape(n, h)
    he = h_E.reshape(n * k, h)
    eidx = E_idx.reshape(n * k)
    ma = mask_attend.reshape(n, k)
    mv = mask_V.reshape(n, 1)

    wb = lambda v: v.astype(jnp.bfloat16)
    w1a, w1b, w1c = wb(W1_w[:h]), wb(W1_w[h:2 * h]), wb(W1_w[2 * h:])
    w11a, w11b, w11c = wb(W11_w[:h]), wb(W11_w[h:2 * h]), wb(W11_w[2 * h:])
    w2, w3, win, wout = wb(W2_w), wb(W3_w), wb(Win_w), wb(Wout_w)
    w12, w13 = wb(W12_w), wb(W13_w)
    r = lambda v: v.reshape(1, -1)

    a1, p1 = pl.pallas_call(
        _proj_body,
        grid=(grid,),
        in_specs=[
            pl.BlockSpec((nb, h), lambda i: (i, 0)),
            _full((h, h)), _full((h, h)), _full((1, h)),
        ],
        out_specs=[pl.BlockSpec((nb, h), lambda i: (i, 0))] * 2,
        out_shape=[jax.ShapeDtypeStruct((n, h), jnp.float32)] * 2,
        compiler_params=pltpu.CompilerParams(dimension_semantics=("parallel",)),
    )(hv, w1a, w1c, r(W1_b))

    c1 = _sc_gather(p1, eidx)

    hv_new, a2, p2 = pl.pallas_call(
        _enc_body,
        grid=(grid,),
        in_specs=[
            pl.BlockSpec((nb, h), lambda i: (i, 0)),
            pl.BlockSpec((eb, h), lambda i: (i, 0)),
            pl.BlockSpec((eb, h), lambda i: (i, 0)),
            pl.BlockSpec((nb, h), lambda i: (i, 0)),
            pl.BlockSpec((nb, k), lambda i: (i, 0)),
            pl.BlockSpec((nb, 1), lambda i: (i, 0)),
            _full((h, h)), _full((h, h)), _full((1, h)), _full((h, h)), _full((1, h)),
            _full((h, ff)), _full((1, ff)), _full((ff, h)), _full((1, h)),
            _full((1, h)), _full((1, h)), _full((1, h)), _full((1, h)),
            _full((h, h)), _full((1, h)), _full((h, h)),
        ],
        out_specs=[pl.BlockSpec((nb, h), lambda i: (i, 0))] * 3,
        out_shape=[jax.ShapeDtypeStruct((n, h), jnp.float32)] * 3,
        compiler_params=pltpu.CompilerParams(dimension_semantics=("parallel",)),
    )(hv, he, c1, a1, ma, mv,
      w1b, w2, r(W2_b), w3, r(W3_b),
      win, r(Win_b), wout, r(Wout_b),
      r(n1_s), r(n1_o), r(n2_s), r(n2_o),
      w11a, r(W11_b), w11c)

    c2 = _sc_gather(p2, eidx)

    he_new = pl.pallas_call(
        _edge_body,
        grid=(grid,),
        in_specs=[
            pl.BlockSpec((eb, h), lambda i: (i, 0)),
            pl.BlockSpec((eb, h), lambda i: (i, 0)),
            pl.BlockSpec((nb, h), lambda i: (i, 0)),
            _full((h, h)), _full((h, h)), _full((1, h)), _full((h, h)), _full((1, h)),
            _full((1, h)), _full((1, h)),
        ],
        out_specs=pl.BlockSpec((eb, h), lambda i: (i, 0)),
        out_shape=jax.ShapeDtypeStruct((n * k, h), jnp.float32),
        compiler_params=pltpu.CompilerParams(dimension_semantics=("parallel",)),
    )(he, c2, a2,
      w11b, w12, r(W12_b), w13, r(W13_b), r(n3_s), r(n3_o))

    return hv_new.reshape(b, n, h), he_new.reshape(b, n, k, h)


# speedup vs baseline: 1.2182x; 1.2182x over previous
import functools

import jax
import jax.numpy as jnp
from jax import lax
from jax.experimental import pallas as pl
from jax.experimental.pallas import tpu as pltpu
from jax.experimental.pallas import tpu_sc as plsc


_NW = 32
_SUB = 128
_NSUB = 5
_SUPER = _SUB * _NSUB


def _sc_gather(table, idx1d):
    nrows = idx1d.shape[0]
    assert nrows % _SUPER == 0
    h = table.shape[1]
    nsuper = nrows // _SUPER
    per = (nsuper + _NW - 1) // _NW
    mesh = plsc.VectorSubcoreMesh(core_axis_name="c", subcore_axis_name="s")

    @functools.partial(
        pl.kernel,
        mesh=mesh,
        out_type=jax.ShapeDtypeStruct((nrows, h), table.dtype),
        scratch_types=[
            pltpu.VMEM((_SUPER,), jnp.int32),
            pltpu.VMEM((_SUPER, h), table.dtype),
            pltpu.SemaphoreType.DMA,
        ],
    )
    def gk(table_hbm, idx_hbm, out_hbm, idx_v, rows_v, sem):
        wid = lax.axis_index("s") * 2 + lax.axis_index("c")

        def body(j, carry):
            sc = j * _NW + wid

            @pl.when(sc < nsuper)
            def _():
                pltpu.sync_copy(idx_hbm.at[pl.ds(sc * _SUPER, _SUPER)], idx_v)
                copies = [
                    pltpu.async_copy(
                        table_hbm.at[idx_v.at[pl.ds(i * _SUB, _SUB)]],
                        rows_v.at[pl.ds(i * _SUB, _SUB)],
                        sem,
                    )
                    for i in range(_NSUB)
                ]
                for cp in copies:
                    cp.wait()
                pltpu.sync_copy(rows_v, out_hbm.at[pl.ds(sc * _SUPER, _SUPER)])

            return carry

        lax.fori_loop(0, per, body, 0)

    return gk(table, idx1d)


_NB = 400


def _gelu(x):
    return 0.5 * x * (1.0 + lax.erf(x * 0.7071067811865476))


def _bdot(x, w):
    return jnp.dot(x.astype(jnp.bfloat16), w, preferred_element_type=jnp.float32)


def _ln(x, s, o):
    mu = jnp.mean(x, axis=-1, keepdims=True)
    xc = x - mu
    var = jnp.mean(xc * xc, axis=-1, keepdims=True)
    return s * xc * lax.rsqrt(var + 1e-5) + o


def _proj_body(hv_ref, w1a_ref, w1c_ref, b1_ref, a_ref, p_ref):
    hv = hv_ref[...]
    a_ref[...] = _bdot(hv, w1a_ref[...]) + b1_ref[...]
    p_ref[...] = _bdot(hv, w1c_ref[...])


def _enc_body(hv_ref, he_ref, c1_ref, a1_ref, ma_ref, mv_ref,
              w1b_ref, w2_ref, b2_ref, w3_ref, b3_ref,
              win_ref, bin_ref, wout_ref, bout_ref,
              n1s_ref, n1o_ref, n2s_ref, n2o_ref,
              w11a_ref, b11_ref, w11c_ref,
              hvout_ref, a2_ref, p2_ref):
    hv = hv_ref[...]
    he = he_ref[...]
    nb, h = hv.shape
    eb = he.shape[0]
    k = eb // nb

    x = _bdot(he, w1b_ref[...]) + c1_ref[...].astype(jnp.float32)
    x = x.reshape(nb, k, h) + a1_ref[...][:, None, :]
    m = _gelu(x).reshape(eb, h)
    m = _gelu(_bdot(m, w2_ref[...]) + b2_ref[...])
    m = _bdot(m, w3_ref[...]) + b3_ref[...]
    m = m.reshape(nb, k, h) * ma_ref[...][:, :, None]
    dh = jnp.sum(m, axis=1) * (1.0 / 30.0)

    h1 = _ln(hv + dh, n1s_ref[...], n1o_ref[...])
    f = _gelu(_bdot(h1, win_ref[...]) + bin_ref[...])
    f = _bdot(f, wout_ref[...]) + bout_ref[...]
    h2 = _ln(h1 + f, n2s_ref[...], n2o_ref[...])
    hv_new = mv_ref[...] * h2

    hvout_ref[...] = hv_new
    a2_ref[...] = _bdot(hv_new, w11a_ref[...]) + b11_ref[...]
    p2_ref[...] = _bdot(hv_new, w11c_ref[...])


def _edge_body(he_ref, c2_ref, a2_ref,
               w11b_ref, w12_ref, b12_ref, w13_ref, b13_ref,
               n3s_ref, n3o_ref, heout_ref):
    he = he_ref[...]
    eb, h = he.shape
    a2 = a2_ref[...]
    nb = a2.shape[0]
    k = eb // nb

    x = _bdot(he, w11b_ref[...]) + c2_ref[...].astype(jnp.float32)
    x = x.reshape(nb, k, h) + a2[:, None, :]
    m = _gelu(x).reshape(eb, h)
    m = _gelu(_bdot(m, w12_ref[...]) + b12_ref[...])
    m = _bdot(m, w13_ref[...]) + b13_ref[...]
    heout_ref[...] = _ln(he + m, n3s_ref[...], n3o_ref[...])


def _full(shape):
    return pl.BlockSpec(shape, lambda i: (0, 0))


@jax.jit
def kernel(h_V, h_E, mask_V, mask_attend,
           W1_w, W1_b, W2_w, W2_b, W3_w, W3_b,
           W11_w, W11_b, W12_w, W12_b, W13_w, W13_b,
           Win_w, Win_b, Wout_w, Wout_b,
           n1_s, n1_o, n2_s, n2_o, n3_s, n3_o, E_idx):
    b, n, h = h_V.shape
    k = E_idx.shape[-1]
    ff = Win_w.shape[1]
    nb = _NB
    grid = n // nb
    eb = nb * k

    hv = h_V.reshape(n, h)
    he = h_E.reshape(n * k, h)
    eidx = E_idx.reshape(n * k)
    ma = mask_attend.reshape(n, k)
    mv = mask_V.reshape(n, 1)

    wb = lambda v: v.astype(jnp.bfloat16)
    w1a, w1b, w1c = wb(W1_w[:h]), wb(W1_w[h:2 * h]), wb(W1_w[2 * h:])
    w11a, w11b, w11c = wb(W11_w[:h]), wb(W11_w[h:2 * h]), wb(W11_w[2 * h:])
    w2, w3, win, wout = wb(W2_w), wb(W3_w), wb(Win_w), wb(Wout_w)
    w12, w13 = wb(W12_w), wb(W13_w)
    r = lambda v: v.reshape(1, -1)

    a1, p1 = pl.pallas_call(
        _proj_body,
        grid=(grid,),
        in_specs=[
            pl.BlockSpec((nb, h), lambda i: (i, 0)),
            _full((h, h)), _full((h, h)), _full((1, h)),
        ],
        out_specs=[pl.BlockSpec((nb, h), lambda i: (i, 0))] * 2,
        out_shape=[jax.ShapeDtypeStruct((n, h), jnp.float32)] * 2,
        compiler_params=pltpu.CompilerParams(dimension_semantics=("parallel",)),
    )(hv, w1a, w1c, r(W1_b))

    c1 = _sc_gather(p1, eidx)

    hv_new, a2, p2 = pl.pallas_call(
        _enc_body,
        grid=(grid,),
        in_specs=[
            pl.BlockSpec((nb, h), lambda i: (i, 0)),
            pl.BlockSpec((eb, h), lambda i: (i, 0)),
            pl.BlockSpec((eb, h), lambda i: (i, 0)),
            pl.BlockSpec((nb, h), lambda i: (i, 0)),
            pl.BlockSpec((nb, k), lambda i: (i, 0)),
            pl.BlockSpec((nb, 1), lambda i: (i, 0)),
            _full((h, h)), _full((h, h)), _full((1, h)), _full((h, h)), _full((1, h)),
            _full((h, ff)), _full((1, ff)), _full((ff, h)), _full((1, h)),
            _full((1, h)), _full((1, h)), _full((1, h)), _full((1, h)),
            _full((h, h)), _full((1, h)), _full((h, h)),
        ],
        out_specs=[pl.BlockSpec((nb, h), lambda i: (i, 0))] * 3,
        out_shape=[jax.ShapeDtypeStruct((n, h), jnp.float32)] * 3,
        compiler_params=pltpu.CompilerParams(dimension_semantics=("parallel",)),
    )(hv, he, c1, a1, ma, mv,
      w1b, w2, r(W2_b), w3, r(W3_b),
      win, r(Win_b), wout, r(Wout_b),
      r(n1_s), r(n1_o), r(n2_s), r(n2_o),
      w11a, r(W11_b), w11c)

    c2 = _sc_gather(p2, eidx)

    he_new = pl.pallas_call(
        _edge_body,
        grid=(grid,),
        in_specs=[
            pl.BlockSpec((eb, h), lambda i: (i, 0)),
            pl.BlockSpec((eb, h), lambda i: (i, 0)),
            pl.BlockSpec((nb, h), lambda i: (i, 0)),
            _full((h, h)), _full((h, h)), _full((1, h)), _full((h, h)), _full((1, h)),
            _full((1, h)), _full((1, h)),
        ],
        out_specs=pl.BlockSpec((eb, h), lambda i: (i, 0)),
        out_shape=jax.ShapeDtypeStruct((n * k, h), jnp.float32),
        compiler_params=pltpu.CompilerParams(dimension_semantics=("parallel",)),
    )(he, c2, a2,
      w11b, w12, r(W12_b), w13, r(W13_b), r(n3_s), r(n3_o))

    return hv_new.reshape(b, n, h), he_new.reshape(b, n, k, h)
